# baseline (speedup 1.0000x reference)
import functools

import jax
import jax.numpy as jnp
from jax import lax
from jax.experimental import pallas as pl
from jax.experimental.pallas import tpu as pltpu
from jax.experimental.pallas import tpu_sc as plsc

BATCH = 16384
EMBED_DIM = 32


def _gather_kernel(table_hbm, idx_hbm, out_hbm, idx_v, rows_v, sem):
    info = plsc.get_sparse_core_info()
    num_cores = info.num_cores
    b_per_w = BATCH // (num_cores * info.num_subcores)
    wid = lax.axis_index("s") * num_cores + lax.axis_index("c")
    base = wid * b_per_w
    pltpu.sync_copy(idx_hbm.at[pl.ds(base, b_per_w)], idx_v)
    pltpu.async_copy(table_hbm.at[idx_v], rows_v, sem).wait()
    pltpu.sync_copy(rows_v, out_hbm.at[pl.ds(base, b_per_w)])


def kernel(inputs, table):
    info = plsc.get_sparse_core_info()
    num_workers = info.num_cores * info.num_subcores
    b_per_w = BATCH // num_workers
    mesh = plsc.VectorSubcoreMesh(core_axis_name="c", subcore_axis_name="s")
    run = functools.partial(
        pl.kernel,
        mesh=mesh,
        out_type=jax.ShapeDtypeStruct((BATCH, EMBED_DIM), jnp.float32),
        scratch_types=[
            pltpu.VMEM((b_per_w,), jnp.int32),
            pltpu.VMEM((b_per_w, EMBED_DIM), jnp.float32),
            pltpu.SemaphoreType.DMA,
        ],
        compiler_params=pltpu.CompilerParams(use_tc_tiling_on_sc=False),
    )(_gather_kernel)
    return run(table, inputs.astype(jnp.int32))


# speedup vs baseline: 1.0620x; 1.0620x over previous
import functools

import jax
import jax.numpy as jnp
from jax import lax
from jax.experimental import pallas as pl
from jax.experimental.pallas import tpu as pltpu
from jax.experimental.pallas import tpu_sc as plsc

BATCH = 16384
EMBED_DIM = 32
NUM_WORKERS = 32
B_PER_W = BATCH // NUM_WORKERS
K = 16


def _gather_kernel(table_hbm, idx_hbm, out_hbm, idx_v, rows_v, sem):
    num_cores = plsc.get_sparse_core_info().num_cores
    wid = lax.axis_index("s") * num_cores + lax.axis_index("c")
    base = wid * B_PER_W
    pltpu.sync_copy(idx_hbm.at[pl.ds(base, B_PER_W)], idx_v)

    def group(g, _):
        idx_vec = idx_v[pl.ds(g * K, K)]
        for j in range(K):
            r = idx_vec[j]
            pltpu.async_copy(
                table_hbm.at[pl.ds(r, 1), :], rows_v.at[pl.ds(g * K + j, 1), :], sem
            )
        for j in range(K):
            pltpu.make_async_copy(
                table_hbm.at[pl.ds(0, 1), :], rows_v.at[pl.ds(g * K + j, 1), :], sem
            ).wait()
        return ()

    lax.fori_loop(0, B_PER_W // K, group, (), unroll=False)
    pltpu.sync_copy(rows_v, out_hbm.at[pl.ds(base, B_PER_W)])


def kernel(inputs, table):
    mesh = plsc.VectorSubcoreMesh(core_axis_name="c", subcore_axis_name="s")
    run = functools.partial(
        pl.kernel,
        mesh=mesh,
        out_type=jax.ShapeDtypeStruct((BATCH, EMBED_DIM), jnp.float32),
        scratch_types=[
            pltpu.VMEM((B_PER_W,), jnp.int32),
            pltpu.VMEM((B_PER_W, EMBED_DIM), jnp.float32),
            pltpu.SemaphoreType.DMA,
        ],
    )(_gather_kernel)
    return run(table, inputs.astype(jnp.int32))


# speedup vs baseline: 2.1854x; 2.0578x over previous
import functools

import jax
import jax.numpy as jnp
from jax import lax
from jax.experimental import pallas as pl
from jax.experimental.pallas import tpu as pltpu
from jax.experimental.pallas import tpu_sc as plsc

BATCH = 16384
EMBED_DIM = 32
VOCAB_ROWS = 100001
CHUNK = 8192
LANES = 16


def _gather_kernel(table_hbm, idx_hbm, out_hbm, row_v, idx_v, out_v):
    num_cores = plsc.get_sparse_core_info().num_cores
    w = lax.axis_index("s") * num_cores + lax.axis_index("c")
    pltpu.sync_copy(table_hbm.at[w], row_v)

    def chunk_body(c, _):
        base = c * CHUNK
        pltpu.sync_copy(idx_hbm.at[pl.ds(base, CHUNK)], idx_v)

        def vec_body(i, _):
            idx = idx_v[pl.ds(i * LANES, LANES)]
            out_v[pl.ds(i * LANES, LANES)] = plsc.load_gather(row_v, [idx])
            return ()

        lax.fori_loop(0, CHUNK // LANES, vec_body, (), unroll=8)
        pltpu.sync_copy(out_v, out_hbm.at[w, pl.ds(base, CHUNK)])
        return ()

    lax.fori_loop(0, BATCH // CHUNK, chunk_body, ())


def kernel(inputs, table):
    mesh = plsc.VectorSubcoreMesh(core_axis_name="c", subcore_axis_name="s")
    run = functools.partial(
        pl.kernel,
        mesh=mesh,
        out_type=jax.ShapeDtypeStruct((EMBED_DIM, BATCH), jnp.float32),
        scratch_types=[
            pltpu.VMEM((VOCAB_ROWS,), jnp.float32),
            pltpu.VMEM((CHUNK,), jnp.int32),
            pltpu.VMEM((CHUNK,), jnp.float32),
        ],
        compiler_params=pltpu.CompilerParams(needs_layout_passes=False),
    )(_gather_kernel)
    out_t = run(table.T, inputs.astype(jnp.int32))
    return out_t.T


# speedup vs baseline: 2.6530x; 1.2140x over previous
import functools

import jax
import jax.numpy as jnp
from jax import lax
from jax.experimental import pallas as pl
from jax.experimental.pallas import tpu as pltpu
from jax.experimental.pallas import tpu_sc as plsc

BATCH = 16384
EMBED_DIM = 32
VOCAB_ROWS = 100001
CHUNK = 8192
LANES = 16


def _gather_kernel(table_hbm, idx_hbm, out_hbm, row_v, idx_v, out_v):
    num_cores = plsc.get_sparse_core_info().num_cores
    w = lax.axis_index("s") * num_cores + lax.axis_index("c")
    pltpu.sync_copy(table_hbm.at[w], row_v)

    def chunk_body(c, _):
        base = c * CHUNK
        pltpu.sync_copy(idx_hbm.at[pl.ds(base, CHUNK)], idx_v)

        @plsc.parallel_loop(0, CHUNK, LANES, unroll=8)
        def _(i):
            idx = idx_v[pl.ds(i, LANES)]
            out_v[pl.ds(i, LANES)] = plsc.load_gather(row_v, [idx])
        pltpu.sync_copy(out_v, out_hbm.at[w, pl.ds(base, CHUNK)])
        return ()

    lax.fori_loop(0, BATCH // CHUNK, chunk_body, ())


def kernel(inputs, table):
    mesh = plsc.VectorSubcoreMesh(core_axis_name="c", subcore_axis_name="s")
    run = functools.partial(
        pl.kernel,
        mesh=mesh,
        out_type=jax.ShapeDtypeStruct((EMBED_DIM, BATCH), jnp.float32),
        scratch_types=[
            pltpu.VMEM((VOCAB_ROWS,), jnp.float32),
            pltpu.VMEM((CHUNK,), jnp.int32),
            pltpu.VMEM((CHUNK,), jnp.float32),
        ],
        compiler_params=pltpu.CompilerParams(needs_layout_passes=False),
    )(_gather_kernel)
    out_t = run(table.T, inputs.astype(jnp.int32))
    return out_t.T


# speedup vs baseline: 2.7108x; 1.0218x over previous
import functools

import jax
import jax.numpy as jnp
from jax import lax
from jax.experimental import pallas as pl
from jax.experimental.pallas import tpu as pltpu
from jax.experimental.pallas import tpu_sc as plsc

BATCH = 16384
EMBED_DIM = 32
VOCAB_ROWS = 100001
CHUNK = 4096
NCHUNK = BATCH // CHUNK
LANES = 16


def _gather_kernel(
    table_hbm,
    idx_hbm,
    out_hbm,
    row_v,
    idx_a,
    idx_b,
    out_a,
    out_b,
    sem_row,
    sem_idx,
    sem_out,
):
    num_cores = plsc.get_sparse_core_info().num_cores
    w = lax.axis_index("s") * num_cores + lax.axis_index("c")
    idx_bufs = [idx_a, idx_b]
    out_bufs = [out_a, out_b]

    row_copy = pltpu.async_copy(table_hbm.at[w], row_v, sem_row)
    idx_copies = [
        pltpu.async_copy(
            idx_hbm.at[pl.ds(c * CHUNK, CHUNK)], idx_bufs[c], sem_idx.at[c]
        )
        for c in range(2)
    ]
    row_copy.wait()

    out_copies = [None, None]
    for c in range(NCHUNK):
        idx_copies[c % 2].wait()
        if c + 2 < NCHUNK:
            idx_copies[c % 2] = pltpu.async_copy(
                idx_hbm.at[pl.ds((c + 2) * CHUNK, CHUNK)],
                idx_bufs[c % 2],
                sem_idx.at[c % 2],
            )
        if out_copies[c % 2] is not None:
            out_copies[c % 2].wait()

        idx_ref = idx_bufs[c % 2]
        out_ref = out_bufs[c % 2]

        @plsc.parallel_loop(0, CHUNK, LANES, unroll=8)
        def _(i):
            idx = idx_ref[pl.ds(i, LANES)]
            out_ref[pl.ds(i, LANES)] = plsc.load_gather(row_v, [idx])

        out_copies[c % 2] = pltpu.async_copy(
            out_bufs[c % 2], out_hbm.at[w, pl.ds(c * CHUNK, CHUNK)], sem_out.at[c % 2]
        )
    for c in range(2):
        out_copies[c].wait()


def kernel(inputs, table):
    mesh = plsc.VectorSubcoreMesh(core_axis_name="c", subcore_axis_name="s")
    run = functools.partial(
        pl.kernel,
        mesh=mesh,
        out_type=jax.ShapeDtypeStruct((EMBED_DIM, BATCH), jnp.float32),
        scratch_types=[
            pltpu.VMEM((VOCAB_ROWS,), jnp.float32),
            pltpu.VMEM((CHUNK,), jnp.int32),
            pltpu.VMEM((CHUNK,), jnp.int32),
            pltpu.VMEM((CHUNK,), jnp.float32),
            pltpu.VMEM((CHUNK,), jnp.float32),
            pltpu.SemaphoreType.DMA,
            pltpu.SemaphoreType.DMA((2,)),
            pltpu.SemaphoreType.DMA((2,)),
        ],
        compiler_params=pltpu.CompilerParams(needs_layout_passes=False),
    )(_gather_kernel)
    out_t = run(table.T, inputs.astype(jnp.int32))
    return out_t.T
